# baseline (speedup 1.0000x reference)
import functools

import jax
import jax.numpy as jnp
from jax import lax
from jax.experimental import pallas as pl
from jax.experimental.pallas import tpu as pltpu
from jax.experimental.pallas import tpu_sc as plsc

BETA = 0.25


def _argmin_body(K, KC, KS, carry_bf16, r_ref, cb_ref, idx_ref, bd_ref, bi_ref):
    j = pl.program_id(1)
    r = r_ref[...]
    zz = jnp.sum(r * r, axis=1, keepdims=True)
    best = []
    for s in range(KC // KS):
        cb = cb_ref[pl.ds(s * KS, KS), :]
        cc = jnp.sum(cb * cb, axis=1)
        mm = lax.dot_general(r, cb, (((1,), (1,)), ((), ())),
                             preferred_element_type=jnp.float32)
        e = (zz + cc[None, :]) - 2.0 * mm
        m = jnp.min(e, axis=1, keepdims=True)
        ii = lax.broadcasted_iota(jnp.int32, e.shape, 1)
        li = (jnp.min(jnp.where(e == m, ii, K), axis=1, keepdims=True)
              + (j * KC + s * KS))
        best.append((m, li))
    while len(best) > 1:
        nxt = []
        for a in range(0, len(best) - 1, 2):
            (ma, ia), (mb, ib) = best[a], best[a + 1]
            b = mb < ma
            nxt.append((jnp.where(b, mb, ma), jnp.where(b, ib, ia)))
        if len(best) % 2:
            nxt.append(best[-1])
        best = nxt
    m, li = best[0]

    def _round(v):
        if carry_bf16:
            return v.astype(jnp.bfloat16).astype(jnp.float32)
        return v

    @pl.when(j == 0)
    def _():
        bd_ref[...] = _round(m)
        bi_ref[...] = li

    @pl.when(j > 0)
    def _():
        better = m < bd_ref[...]
        bi_ref[...] = jnp.where(better, li, bi_ref[...])
        bd_ref[...] = _round(jnp.where(better, m, bd_ref[...]))

    @pl.when(j == pl.num_programs(1) - 1)
    def _():
        idx_ref[...] = bi_ref[...]


def _layer_argmin(r, cb, carry_bf16):
    B, D = r.shape
    K = cb.shape[0]
    BB = min(128, B)
    KC = min(4096, K)
    KS = min(256, KC)
    out = pl.pallas_call(
        functools.partial(_argmin_body, K, KC, KS, carry_bf16),
        grid=(B // BB, K // KC),
        in_specs=[
            pl.BlockSpec((BB, D), lambda i, j: (i, 0)),
            pl.BlockSpec((KC, D), lambda i, j: (j, 0)),
        ],
        out_specs=pl.BlockSpec((BB, 1), lambda i, j: (i, 0)),
        out_shape=jax.ShapeDtypeStruct((B, 1), jnp.int32),
        scratch_shapes=[
            pltpu.VMEM((BB, 1), jnp.float32),
            pltpu.VMEM((BB, 1), jnp.int32),
        ],
    )(r, cb)
    return out[:, 0]


def _sc_update(cb, idx, r, x=None):
    B, D = r.shape
    NC, NS = 2, 16
    NW = NC * NS
    bw = B // NW
    mesh = plsc.VectorSubcoreMesh(core_axis_name="c", subcore_axis_name="s")

    scratch = [
        pltpu.VMEM((bw,), jnp.int32),
        pltpu.VMEM((bw, D), jnp.float32),
        pltpu.VMEM((bw, D), jnp.float32),
        pltpu.VMEM((16,), jnp.float32),
        pltpu.SemaphoreType.DMA,
    ]
    out_type = (
        jax.ShapeDtypeStruct((B, D), jnp.float32),
        jax.ShapeDtypeStruct((NW, 16), jnp.float32),
    )

    params = pltpu.CompilerParams(use_tc_tiling_on_sc=False)

    if x is None:

        @functools.partial(pl.kernel, out_type=out_type, mesh=mesh,
                           scratch_types=scratch, compiler_params=params)
        def k(cb_hbm, idx_hbm, r_hbm, rout_hbm, loss_hbm,
              idx_v, q_v, r_v, acc_v, sem):
            wid = lax.axis_index("s") * NC + lax.axis_index("c")
            base = wid * bw
            pltpu.sync_copy(idx_hbm.at[pl.ds(base, bw)], idx_v)
            cp = pltpu.async_copy(cb_hbm.at[idx_v], q_v, sem)
            pltpu.sync_copy(r_hbm.at[pl.ds(base, bw)], r_v)
            cp.wait()

            def body(i, acc):
                for j in range(D // 16):
                    q = q_v[i, pl.ds(j * 16, 16)]
                    rr = r_v[i, pl.ds(j * 16, 16)]
                    t = q - rr
                    acc = acc + t * t
                    r_v[i, pl.ds(j * 16, 16)] = rr - (rr + t)
                return acc

            acc = lax.fori_loop(0, bw, body, jnp.zeros((16,), jnp.float32))
            acc_v[...] = acc
            pltpu.sync_copy(r_v, rout_hbm.at[pl.ds(base, bw)])
            pltpu.sync_copy(acc_v, loss_hbm.at[wid])

        return k(cb, idx, r)

    @functools.partial(pl.kernel, out_type=out_type, mesh=mesh,
                       scratch_types=scratch + [pltpu.VMEM((bw, D), jnp.float32)],
                       compiler_params=params)
    def kl(cb_hbm, idx_hbm, r_hbm, x_hbm, yout_hbm, loss_hbm,
           idx_v, q_v, r_v, acc_v, sem, x_v):
        wid = lax.axis_index("s") * NC + lax.axis_index("c")
        base = wid * bw
        pltpu.sync_copy(idx_hbm.at[pl.ds(base, bw)], idx_v)
        cp = pltpu.async_copy(cb_hbm.at[idx_v], q_v, sem)
        pltpu.sync_copy(r_hbm.at[pl.ds(base, bw)], r_v)
        pltpu.sync_copy(x_hbm.at[pl.ds(base, bw)], x_v)
        cp.wait()

        def body(i, acc):
            for j in range(D // 16):
                q = q_v[i, pl.ds(j * 16, 16)]
                rr = r_v[i, pl.ds(j * 16, 16)]
                t = q - rr
                acc = acc + t * t
                r_new = rr - (rr + t)
                r_v[i, pl.ds(j * 16, 16)] = x_v[i, pl.ds(j * 16, 16)] - r_new
            return acc

        acc = lax.fori_loop(0, bw, body, jnp.zeros((16,), jnp.float32))
        acc_v[...] = acc
        pltpu.sync_copy(r_v, yout_hbm.at[pl.ds(base, bw)])
        pltpu.sync_copy(acc_v, loss_hbm.at[wid])

    return kl(cb, idx, r, x)


def kernel(x, codebooks):
    L = codebooks.shape[0]
    B, D = x.shape
    r = x
    idxs = []
    parts = []
    for l in range(L):
        cb = codebooks[l]
        idx = _layer_argmin(r, cb, carry_bf16=(l == 0))
        r, lp = _sc_update(cb, idx, r, x=x if l == L - 1 else None)
        parts.append(lp)
        idxs.append(idx)
    y = r
    total = (1.0 + BETA) * jnp.sum(jnp.stack(parts)) / (B * D)
    return y, jnp.stack(idxs, axis=-1), total / L


# speedup vs baseline: 2.2264x; 2.2264x over previous
import functools

import jax
import jax.numpy as jnp
from jax import lax
from jax.experimental import pallas as pl
from jax.experimental.pallas import tpu as pltpu
from jax.experimental.pallas import tpu_sc as plsc

BETA = 0.25


def _argmin_body(K, KC, KS, carry_bf16, r_ref, cb_ref, idx_ref, bd_ref, bi_ref):
    j = pl.program_id(1)
    r = r_ref[...]
    zz = jnp.sum(r * r, axis=1, keepdims=True)
    best = []
    for s in range(KC // KS):
        cb = cb_ref[pl.ds(s * KS, KS), :]
        cc = jnp.sum(cb * cb, axis=1)
        mm = lax.dot_general(r, cb, (((1,), (1,)), ((), ())),
                             preferred_element_type=jnp.float32)
        e = (zz + cc[None, :]) - 2.0 * mm
        m = jnp.min(e, axis=1, keepdims=True)
        ii = lax.broadcasted_iota(jnp.int32, e.shape, 1)
        li = (jnp.min(jnp.where(e == m, ii, K), axis=1, keepdims=True)
              + (j * KC + s * KS))
        best.append((m, li))
    while len(best) > 1:
        nxt = []
        for a in range(0, len(best) - 1, 2):
            (ma, ia), (mb, ib) = best[a], best[a + 1]
            b = mb < ma
            nxt.append((jnp.where(b, mb, ma), jnp.where(b, ib, ia)))
        if len(best) % 2:
            nxt.append(best[-1])
        best = nxt
    m, li = best[0]

    def _round(v):
        if carry_bf16:
            return v.astype(jnp.bfloat16).astype(jnp.float32)
        return v

    @pl.when(j == 0)
    def _():
        bd_ref[...] = _round(m)
        bi_ref[...] = li

    @pl.when(j > 0)
    def _():
        better = m < bd_ref[...]
        bi_ref[...] = jnp.where(better, li, bi_ref[...])
        bd_ref[...] = _round(jnp.where(better, m, bd_ref[...]))

    @pl.when(j == pl.num_programs(1) - 1)
    def _():
        idx_ref[...] = bi_ref[...]


def _layer_argmin(r, cb, carry_bf16):
    B, D = r.shape
    K = cb.shape[0]
    BB = min(512, B)
    KC = min(4096, K)
    KS = min(256, KC)
    out = pl.pallas_call(
        functools.partial(_argmin_body, K, KC, KS, carry_bf16),
        grid=(B // BB, K // KC),
        in_specs=[
            pl.BlockSpec((BB, D), lambda i, j: (i, 0)),
            pl.BlockSpec((KC, D), lambda i, j: (j, 0)),
        ],
        out_specs=pl.BlockSpec((BB, 1), lambda i, j: (i, 0)),
        out_shape=jax.ShapeDtypeStruct((B, 1), jnp.int32),
        scratch_shapes=[
            pltpu.VMEM((BB, 1), jnp.float32),
            pltpu.VMEM((BB, 1), jnp.int32),
        ],
    )(r, cb)
    return out[:, 0]


def _sc_update(cb, idx, r, x=None):
    B, D = r.shape
    NC, NS = 2, 16
    NW = NC * NS
    bw = B // NW
    mesh = plsc.VectorSubcoreMesh(core_axis_name="c", subcore_axis_name="s")

    scratch = [
        pltpu.VMEM((bw,), jnp.int32),
        pltpu.VMEM((bw, D), jnp.float32),
        pltpu.VMEM((bw, D), jnp.float32),
        pltpu.VMEM((16,), jnp.float32),
        pltpu.SemaphoreType.DMA,
    ]
    out_type = (
        jax.ShapeDtypeStruct((B, D), jnp.float32),
        jax.ShapeDtypeStruct((NW, 16), jnp.float32),
    )

    params = pltpu.CompilerParams(use_tc_tiling_on_sc=False)

    if x is None:

        @functools.partial(pl.kernel, out_type=out_type, mesh=mesh,
                           scratch_types=scratch, compiler_params=params)
        def k(cb_hbm, idx_hbm, r_hbm, rout_hbm, loss_hbm,
              idx_v, q_v, r_v, acc_v, sem):
            wid = lax.axis_index("s") * NC + lax.axis_index("c")
            base = wid * bw
            pltpu.sync_copy(idx_hbm.at[pl.ds(base, bw)], idx_v)
            cp = pltpu.async_copy(cb_hbm.at[idx_v], q_v, sem)
            pltpu.sync_copy(r_hbm.at[pl.ds(base, bw)], r_v)
            cp.wait()

            def body(i, acc):
                for j in range(D // 16):
                    q = q_v[i, pl.ds(j * 16, 16)]
                    rr = r_v[i, pl.ds(j * 16, 16)]
                    t = q - rr
                    acc = acc + t * t
                    r_v[i, pl.ds(j * 16, 16)] = rr - (rr + t)
                return acc

            acc = lax.fori_loop(0, bw, body, jnp.zeros((16,), jnp.float32))
            acc_v[...] = acc
            pltpu.sync_copy(r_v, rout_hbm.at[pl.ds(base, bw)])
            pltpu.sync_copy(acc_v, loss_hbm.at[wid])

        return k(cb, idx, r)

    @functools.partial(pl.kernel, out_type=out_type, mesh=mesh,
                       scratch_types=scratch + [pltpu.VMEM((bw, D), jnp.float32)],
                       compiler_params=params)
    def kl(cb_hbm, idx_hbm, r_hbm, x_hbm, yout_hbm, loss_hbm,
           idx_v, q_v, r_v, acc_v, sem, x_v):
        wid = lax.axis_index("s") * NC + lax.axis_index("c")
        base = wid * bw
        pltpu.sync_copy(idx_hbm.at[pl.ds(base, bw)], idx_v)
        cp = pltpu.async_copy(cb_hbm.at[idx_v], q_v, sem)
        pltpu.sync_copy(r_hbm.at[pl.ds(base, bw)], r_v)
        pltpu.sync_copy(x_hbm.at[pl.ds(base, bw)], x_v)
        cp.wait()

        def body(i, acc):
            for j in range(D // 16):
                q = q_v[i, pl.ds(j * 16, 16)]
                rr = r_v[i, pl.ds(j * 16, 16)]
                t = q - rr
                acc = acc + t * t
                r_new = rr - (rr + t)
                r_v[i, pl.ds(j * 16, 16)] = x_v[i, pl.ds(j * 16, 16)] - r_new
            return acc

        acc = lax.fori_loop(0, bw, body, jnp.zeros((16,), jnp.float32))
        acc_v[...] = acc
        pltpu.sync_copy(r_v, yout_hbm.at[pl.ds(base, bw)])
        pltpu.sync_copy(acc_v, loss_hbm.at[wid])

    return kl(cb, idx, r, x)


def kernel(x, codebooks):
    L = codebooks.shape[0]
    B, D = x.shape
    r = x
    idxs = []
    parts = []
    for l in range(L):
        cb = codebooks[l]
        idx = _layer_argmin(r, cb, carry_bf16=(l == 0))
        r, lp = _sc_update(cb, idx, r, x=x if l == L - 1 else None)
        parts.append(lp)
        idxs.append(idx)
    y = r
    total = (1.0 + BETA) * jnp.sum(jnp.stack(parts)) / (B * D)
    return y, jnp.stack(idxs, axis=-1), total / L
